# Initial kernel scaffold; baseline (speedup 1.0000x reference)
import jax
import jax.numpy as jnp
from jax.experimental import pallas as pl


def kernel(seq, W, K):
    raise NotImplementedError("write your pallas kernel here")


# speedup vs baseline: 44.4027x; 44.4027x over previous
import functools

import jax
import jax.numpy as jnp
from jax import lax
from jax.experimental import pallas as pl
from jax.experimental.pallas import tpu as pltpu
from jax.experimental.pallas import tpu_sc as plsc

_VOCAB = 100000
_EMB = 32
_MAXW = 7
_B = 1024
_L = 200
_NCORES = 2
_NSUB = 16
_NW = _NCORES * _NSUB
_ROWS_PER_W = _B // _NW
_CHUNKS = ((0, 128), (128, 72))
_L3 = _L - 2
_L5 = _L - 4
_L7 = _L - 6


def _sc_body(seq_hbm, w_hbm, k_hbm, o3_hbm, o5_hbm, o7_hbm,
             idx_v, wrows_v, kslab_v, o3_v, o5_v, o7_v, sem):
    cid = lax.axis_index("c")
    sid = lax.axis_index("s")
    wid = sid * _NCORES + cid

    def _p(c, d, h):
        kv = kslab_v[c, pl.ds((3 + d) * _EMB + h * 16, 16)]
        wv = wrows_v[c + d, pl.ds(h * 16, 16)]
        return wv * kv

    def row_body(ri, carry):
        b = wid * _ROWS_PER_W + ri
        pltpu.sync_copy(seq_hbm.at[b], idx_v)
        copies = []
        for off, n in _CHUNKS:
            sl = pl.ds(off, n)
            copies.append(pltpu.async_copy(
                w_hbm.at[idx_v.at[sl]], wrows_v.at[sl], sem))
            copies.append(pltpu.async_copy(
                k_hbm.at[idx_v.at[sl]], kslab_v.at[sl], sem))
        for cp in copies:
            cp.wait()

        def c_body(c, inner):
            for h in range(2):
                hs = pl.ds(h * 16, 16)
                p0 = _p(c, 0, h)
                m3 = jnp.maximum(jnp.maximum(_p(c, -1, h), p0), _p(c, 1, h))
                m5 = jnp.maximum(m3, jnp.maximum(_p(c, -2, h), _p(c, 2, h)))
                m7 = jnp.maximum(m5, jnp.maximum(_p(c, -3, h), _p(c, 3, h)))
                o3_v[c - 1, hs] = m3
                o5_v[c - 2, hs] = m5
                o7_v[c - 3, hs] = m7
            return inner

        lax.fori_loop(3, _L - 3, c_body, None)

        for c in (1, 2, _L - 3, _L - 2):
            for h in range(2):
                hs = pl.ds(h * 16, 16)
                p0 = _p(c, 0, h)
                m3 = jnp.maximum(jnp.maximum(_p(c, -1, h), p0), _p(c, 1, h))
                o3_v[c - 1, hs] = m3
                if 2 <= c <= _L - 3:
                    m5 = jnp.maximum(
                        m3, jnp.maximum(_p(c, -2, h), _p(c, 2, h)))
                    o5_v[c - 2, hs] = m5

        pltpu.sync_copy(o3_v, o3_hbm.at[b])
        pltpu.sync_copy(o5_v, o5_hbm.at[b])
        pltpu.sync_copy(o7_v, o7_hbm.at[b])
        return carry

    lax.fori_loop(0, _ROWS_PER_W, row_body, None)


@jax.jit
def _impl(seq, W, K):
    seq2 = seq.astype(jnp.int32)
    K2 = K.reshape(_VOCAB, _MAXW * _EMB)
    mesh = plsc.VectorSubcoreMesh(core_axis_name="c", subcore_axis_name="s")
    run = pl.kernel(
        _sc_body,
        mesh=mesh,
        compiler_params=pltpu.CompilerParams(use_tc_tiling_on_sc=False),
        out_type=(
            jax.ShapeDtypeStruct((_B, _L3, _EMB), jnp.float32),
            jax.ShapeDtypeStruct((_B, _L5, _EMB), jnp.float32),
            jax.ShapeDtypeStruct((_B, _L7, _EMB), jnp.float32),
        ),
        scratch_types=[
            pltpu.VMEM((_L,), jnp.int32),
            pltpu.VMEM((_L, _EMB), jnp.float32),
            pltpu.VMEM((_L, _MAXW * _EMB), jnp.float32),
            pltpu.VMEM((_L3, _EMB), jnp.float32),
            pltpu.VMEM((_L5, _EMB), jnp.float32),
            pltpu.VMEM((_L7, _EMB), jnp.float32),
            pltpu.SemaphoreType.DMA,
        ],
    )
    return run(seq2, W, K2)


def kernel(seq, W, K):
    return _impl(seq, W, K)


# speedup vs baseline: 49.2907x; 1.1101x over previous
import jax
import jax.numpy as jnp
from jax import lax
from jax.experimental import pallas as pl
from jax.experimental.pallas import tpu as pltpu
from jax.experimental.pallas import tpu_sc as plsc

_VOCAB = 100000
_EMB = 32
_MAXW = 7
_B = 1024
_L = 200
_NCORES = 2
_NSUB = 16
_NW = _NCORES * _NSUB
_ROWS_PER_W = _B // _NW
_CHUNKS = ((0, 128), (128, 72))
_L3 = _L - 2
_L5 = _L - 4
_L7 = _L - 6


def _sc_body(seq_hbm, w_hbm, k_hbm, o3_hbm, o5_hbm, o7_hbm,
             idx0, idx1, wr0, wr1, ks0, ks1, o3_v, o5_v, o7_v, gsem, ssem):
    cid = lax.axis_index("c")
    sid = lax.axis_index("s")
    wid = sid * _NCORES + cid
    bufs = ((idx0, wr0, ks0), (idx1, wr1, ks1))

    def gather_copies(b, p):
        idx_v, wr, ks = bufs[p]
        cps = []
        for off, n in _CHUNKS:
            sl = pl.ds(off, n)
            cps.append(pltpu.make_async_copy(
                w_hbm.at[idx_v.at[sl]], wr.at[sl], gsem))
            cps.append(pltpu.make_async_copy(
                k_hbm.at[idx_v.at[sl]], ks.at[sl], gsem))
        return cps

    def issue_g(ri, p):
        b = wid * _ROWS_PER_W + ri
        pltpu.sync_copy(seq_hbm.at[b], bufs[p][0])
        for cp in gather_copies(b, p):
            cp.start()

    def wait_g(p):
        for cp in gather_copies(0, p):
            cp.wait()

    def scatter_copies(ri):
        b = wid * _ROWS_PER_W + ri
        return [pltpu.make_async_copy(o3_v, o3_hbm.at[b], ssem),
                pltpu.make_async_copy(o5_v, o5_hbm.at[b], ssem),
                pltpu.make_async_copy(o7_v, o7_hbm.at[b], ssem)]

    def issue_s(ri):
        for cp in scatter_copies(ri):
            cp.start()

    def wait_s():
        for cp in scatter_copies(0):
            cp.wait()

    def compute(p):
        idx_v, wr, ks = bufs[p]

        def _p(c, d, h):
            kv = ks[c, pl.ds((3 + d) * _EMB + h * 16, 16)]
            wv = wr[c + d, pl.ds(h * 16, 16)]
            return wv * kv

        def c_body(c, inner):
            for h in range(2):
                hs = pl.ds(h * 16, 16)
                p0 = _p(c, 0, h)
                m3 = jnp.maximum(jnp.maximum(_p(c, -1, h), p0), _p(c, 1, h))
                m5 = jnp.maximum(m3, jnp.maximum(_p(c, -2, h), _p(c, 2, h)))
                m7 = jnp.maximum(m5, jnp.maximum(_p(c, -3, h), _p(c, 3, h)))
                o3_v[c - 1, hs] = m3
                o5_v[c - 2, hs] = m5
                o7_v[c - 3, hs] = m7
            return inner

        lax.fori_loop(3, _L - 3, c_body, None)

        for c in (1, 2, _L - 3, _L - 2):
            for h in range(2):
                hs = pl.ds(h * 16, 16)
                p0 = _p(c, 0, h)
                m3 = jnp.maximum(jnp.maximum(_p(c, -1, h), p0), _p(c, 1, h))
                o3_v[c - 1, hs] = m3
                if 2 <= c <= _L - 3:
                    m5 = jnp.maximum(
                        m3, jnp.maximum(_p(c, -2, h), _p(c, 2, h)))
                    o5_v[c - 2, hs] = m5

    issue_g(0, 0)
    wait_g(0)
    issue_g(1, 1)
    compute(0)
    issue_s(0)

    def pair_body(i, carry):
        ri = 1 + 2 * i
        for step in range(2):
            p = (1 + step) % 2
            wait_g(p)
            issue_g(ri + step + 1, 1 - p)
            wait_s()
            compute(p)
            issue_s(ri + step)
        return carry

    lax.fori_loop(0, (_ROWS_PER_W - 2) // 2, pair_body, None)

    wait_g(1)
    wait_s()
    compute(1)
    issue_s(_ROWS_PER_W - 1)
    wait_s()


@jax.jit
def _impl(seq, W, K):
    seq2 = seq.astype(jnp.int32)
    K2 = K.reshape(_VOCAB, _MAXW * _EMB)
    mesh = plsc.VectorSubcoreMesh(core_axis_name="c", subcore_axis_name="s")
    run = pl.kernel(
        _sc_body,
        mesh=mesh,
        compiler_params=pltpu.CompilerParams(use_tc_tiling_on_sc=False),
        out_type=(
            jax.ShapeDtypeStruct((_B, _L3, _EMB), jnp.float32),
            jax.ShapeDtypeStruct((_B, _L5, _EMB), jnp.float32),
            jax.ShapeDtypeStruct((_B, _L7, _EMB), jnp.float32),
        ),
        scratch_types=[
            pltpu.VMEM((_L,), jnp.int32),
            pltpu.VMEM((_L,), jnp.int32),
            pltpu.VMEM((_L, _EMB), jnp.float32),
            pltpu.VMEM((_L, _EMB), jnp.float32),
            pltpu.VMEM((_L, _MAXW * _EMB), jnp.float32),
            pltpu.VMEM((_L, _MAXW * _EMB), jnp.float32),
            pltpu.VMEM((_L3, _EMB), jnp.float32),
            pltpu.VMEM((_L5, _EMB), jnp.float32),
            pltpu.VMEM((_L7, _EMB), jnp.float32),
            pltpu.SemaphoreType.DMA,
            pltpu.SemaphoreType.DMA,
        ],
    )
    return run(seq2, W, K2)


def kernel(seq, W, K):
    return _impl(seq, W, K)


# speedup vs baseline: 49.3320x; 1.0008x over previous
import jax
import jax.numpy as jnp
from jax import lax
from jax.experimental import pallas as pl
from jax.experimental.pallas import tpu as pltpu
from jax.experimental.pallas import tpu_sc as plsc

_VOCAB = 100000
_EMB = 32
_MAXW = 7
_B = 1024
_L = 200
_NCORES = 2
_NSUB = 16
_NW = _NCORES * _NSUB
_ROWS_PER_W = _B // _NW
_CHUNKS = ((0, 128), (128, 72))
_L3 = _L - 2
_L5 = _L - 4
_L7 = _L - 6


def _sc_body(seq_hbm, w_hbm, k_hbm, o3_hbm, o5_hbm, o7_hbm,
             idx0, idx1, wr0, wr1, ks0, ks1, o3_v, o5_v, o7_v, gsem, ssem):
    cid = lax.axis_index("c")
    sid = lax.axis_index("s")
    wid = sid * _NCORES + cid
    bufs = ((idx0, wr0, ks0), (idx1, wr1, ks1))

    def gather_copies(b, p):
        idx_v, wr, ks = bufs[p]
        cps = []
        for off, n in _CHUNKS:
            sl = pl.ds(off, n)
            cps.append(pltpu.make_async_copy(
                w_hbm.at[idx_v.at[sl]], wr.at[sl], gsem))
            cps.append(pltpu.make_async_copy(
                k_hbm.at[idx_v.at[sl]], ks.at[sl], gsem))
        return cps

    def issue_g(ri, p):
        b = wid * _ROWS_PER_W + ri
        pltpu.sync_copy(seq_hbm.at[b], bufs[p][0])
        for cp in gather_copies(b, p):
            cp.start()

    def wait_g(p):
        for cp in gather_copies(0, p):
            cp.wait()

    def scatter_copies(ri):
        b = wid * _ROWS_PER_W + ri
        return [pltpu.make_async_copy(o3_v, o3_hbm.at[b], ssem),
                pltpu.make_async_copy(o5_v, o5_hbm.at[b], ssem),
                pltpu.make_async_copy(o7_v, o7_hbm.at[b], ssem)]

    def issue_s(ri):
        for cp in scatter_copies(ri):
            cp.start()

    def wait_s():
        for cp in scatter_copies(0):
            cp.wait()

    def compute(p):
        idx_v, wr, ks = bufs[p]

        def wrow(c, h):
            return wr[c, pl.ds(h * 16, 16)]

        def _p(c, d, h):
            kv = ks[c, pl.ds((3 + d) * _EMB + h * 16, 16)]
            wv = wr[c + d, pl.ds(h * 16, 16)]
            return wv * kv

        init = tuple(wrow(c, h) for c in range(6) for h in range(2))

        @plsc.parallel_loop(3, _L - 3, carry=init, unroll=2)
        def c_body(c, win):
            new = (wrow(c + 3, 0), wrow(c + 3, 1))
            rows = tuple(win[2 * i:2 * i + 2] for i in range(6)) + (new,)
            for h in range(2):
                hs = pl.ds(h * 16, 16)

                def kv(d):
                    return ks[c, pl.ds((3 + d) * _EMB + h * 16, 16)]

                pr = [rows[3 + d][h] * kv(d) for d in range(-3, 4)]
                m3 = jnp.maximum(jnp.maximum(pr[2], pr[3]), pr[4])
                m5 = jnp.maximum(m3, jnp.maximum(pr[1], pr[5]))
                m7 = jnp.maximum(m5, jnp.maximum(pr[0], pr[6]))
                o3_v[c - 1, hs] = m3
                o5_v[c - 2, hs] = m5
                o7_v[c - 3, hs] = m7
            return win[2:] + new

        for c in (1, 2, _L - 3, _L - 2):
            for h in range(2):
                hs = pl.ds(h * 16, 16)
                p0 = _p(c, 0, h)
                m3 = jnp.maximum(jnp.maximum(_p(c, -1, h), p0), _p(c, 1, h))
                o3_v[c - 1, hs] = m3
                if 2 <= c <= _L - 3:
                    m5 = jnp.maximum(
                        m3, jnp.maximum(_p(c, -2, h), _p(c, 2, h)))
                    o5_v[c - 2, hs] = m5

    issue_g(0, 0)
    wait_g(0)
    issue_g(1, 1)
    compute(0)
    issue_s(0)

    def pair_body(i, carry):
        ri = 1 + 2 * i
        for step in range(2):
            p = (1 + step) % 2
            wait_g(p)
            issue_g(ri + step + 1, 1 - p)
            wait_s()
            compute(p)
            issue_s(ri + step)
        return carry

    lax.fori_loop(0, (_ROWS_PER_W - 2) // 2, pair_body, None)

    wait_g(1)
    wait_s()
    compute(1)
    issue_s(_ROWS_PER_W - 1)
    wait_s()


@jax.jit
def _impl(seq, W, K):
    seq2 = seq.astype(jnp.int32)
    K2 = K.reshape(_VOCAB, _MAXW * _EMB)
    mesh = plsc.VectorSubcoreMesh(core_axis_name="c", subcore_axis_name="s")
    run = pl.kernel(
        _sc_body,
        mesh=mesh,
        compiler_params=pltpu.CompilerParams(use_tc_tiling_on_sc=False),
        out_type=(
            jax.ShapeDtypeStruct((_B, _L3, _EMB), jnp.float32),
            jax.ShapeDtypeStruct((_B, _L5, _EMB), jnp.float32),
            jax.ShapeDtypeStruct((_B, _L7, _EMB), jnp.float32),
        ),
        scratch_types=[
            pltpu.VMEM((_L,), jnp.int32),
            pltpu.VMEM((_L,), jnp.int32),
            pltpu.VMEM((_L, _EMB), jnp.float32),
            pltpu.VMEM((_L, _EMB), jnp.float32),
            pltpu.VMEM((_L, _MAXW * _EMB), jnp.float32),
            pltpu.VMEM((_L, _MAXW * _EMB), jnp.float32),
            pltpu.VMEM((_L3, _EMB), jnp.float32),
            pltpu.VMEM((_L5, _EMB), jnp.float32),
            pltpu.VMEM((_L7, _EMB), jnp.float32),
            pltpu.SemaphoreType.DMA,
            pltpu.SemaphoreType.DMA,
        ],
    )
    return run(seq2, W, K2)


def kernel(seq, W, K):
    return _impl(seq, W, K)
